# Initial kernel scaffold; baseline (speedup 1.0000x reference)
import jax
import jax.numpy as jnp
from jax.experimental import pallas as pl


def kernel(idx, targets, wte, wpe):
    raise NotImplementedError("write your pallas kernel here")


# speedup vs baseline: 1.0451x; 1.0451x over previous
import functools

import jax
import jax.numpy as jnp
from jax import lax
from jax.experimental import pallas as pl
from jax.experimental.pallas import tpu as pltpu
from jax.experimental.pallas import tpu_sc as plsc

VOCAB = 50304
D = 768
B, T = 4, 2048
BT = B * T

NC, NS, L = 2, 16, 16
NW = NC * NS
ROWS_PER_W = BT // NW
CHUNK = 64
NCHUNK = ROWS_PER_W // CHUNK


@functools.cache
def _make_kernel():
  mesh = plsc.VectorSubcoreMesh(core_axis_name="c", subcore_axis_name="s")

  @functools.partial(
      pl.kernel,
      mesh=mesh,
      out_type=jax.ShapeDtypeStruct((BT, D), jnp.float32),
      scratch_types=[
          pltpu.VMEM((NCHUNK, CHUNK), jnp.int32),
          pltpu.VMEM((CHUNK, D), jnp.float32),
          pltpu.VMEM((CHUNK, D), jnp.float32),
          pltpu.SemaphoreType.DMA,
      ],
  )
  def emb_kernel(idx_hbm, wte_hbm, wpe_hbm, out_hbm, idx_v, rows_v, wpe_v,
                 sem):
    wid = lax.axis_index("s") * NC + lax.axis_index("c")
    base = wid * ROWS_PER_W
    pos_base = lax.rem(base, T)

    pltpu.sync_copy(idx_hbm.at[pl.ds(wid * NCHUNK, NCHUNK)], idx_v)

    for c in range(NCHUNK):
      row0 = base + c * CHUNK
      gather = pltpu.async_copy(wte_hbm.at[idx_v.at[c]], rows_v, sem)
      pltpu.sync_copy(wpe_hbm.at[pl.ds(pos_base + c * CHUNK, CHUNK)], wpe_v)
      gather.wait()

      def add_row(i, _):
        for j in range(D // L):
          sl = pl.ds(j * L, L)
          rows_v[i, sl] = rows_v[i, sl] + wpe_v[i, sl]
        return 0

      lax.fori_loop(0, CHUNK, add_row, 0)
      pltpu.sync_copy(rows_v, out_hbm.at[pl.ds(row0, CHUNK)])

  return emb_kernel


def kernel(idx, targets, wte, wpe):
  idx2 = idx.astype(jnp.int32).reshape(NW * NCHUNK, CHUNK)
  x = _make_kernel()(idx2, wte, wpe)
  return x.reshape(B, T, D), targets


# speedup vs baseline: 1.0656x; 1.0196x over previous
import functools

import jax
import jax.numpy as jnp
from jax import lax
from jax.experimental import pallas as pl
from jax.experimental.pallas import tpu as pltpu
from jax.experimental.pallas import tpu_sc as plsc

VOCAB = 50304
D = 768
B, T = 4, 2048
BT = B * T

NC, NS, L = 2, 16, 16
NW = NC * NS
T_PER_W = T // NW
CHUNK = 32
SUB = T_PER_W // CHUNK
NCHUNK = B * SUB
NBUF = 3


@functools.cache
def _make_kernel():
  mesh = plsc.VectorSubcoreMesh(core_axis_name="c", subcore_axis_name="s")

  @functools.partial(
      pl.kernel,
      mesh=mesh,
      out_type=jax.ShapeDtypeStruct((BT, D), jnp.float32),
      scratch_types=[
          pltpu.VMEM((NCHUNK, CHUNK), jnp.int32),
          pltpu.VMEM((T_PER_W, D), jnp.float32),
      ] + [pltpu.VMEM((CHUNK, D), jnp.float32) for _ in range(NBUF)]
        + [pltpu.SemaphoreType.DMA for _ in range(2 * NBUF)],
  )
  def emb_kernel(idx_hbm, wte_hbm, wpe_hbm, out_hbm, idx_v, wpe_v,
                 r0, r1, r2, g0, g1, g2, s0, s1, s2):
    rows = (r0, r1, r2)
    gsem = (g0, g1, g2)
    ssem = (s0, s1, s2)
    wid = lax.axis_index("s") * NC + lax.axis_index("c")
    t0 = wid * T_PER_W

    for b in range(B):
      pltpu.sync_copy(idx_hbm.at[pl.ds(b * (T // CHUNK) + wid * SUB, SUB)],
                      idx_v.at[pl.ds(b * SUB, SUB)])
    pltpu.sync_copy(wpe_hbm.at[pl.ds(t0, T_PER_W)], wpe_v)

    def start_gather(k):
      p = k % NBUF
      return pltpu.async_copy(wte_hbm.at[idx_v.at[k]], rows[p], gsem[p])

    gathers = {k: start_gather(k) for k in range(min(NBUF - 1, NCHUNK))}
    stores = {}
    for k in range(NCHUNK):
      p = k % NBUF
      b, s = divmod(k, SUB)
      gathers.pop(k).wait()

      def add_row(i, _, s=s, p=p):
        for j in range(D // L):
          sl = pl.ds(j * L, L)
          rows[p][i, sl] = rows[p][i, sl] + wpe_v[s * CHUNK + i, sl]
        return 0

      lax.fori_loop(0, CHUNK, add_row, 0)
      row0 = b * T + t0 + s * CHUNK
      stores[k] = pltpu.async_copy(rows[p], out_hbm.at[pl.ds(row0, CHUNK)],
                                   ssem[p])
      nxt = k + NBUF - 1
      if nxt < NCHUNK:
        prev = nxt - NBUF
        if prev >= 0:
          stores.pop(prev).wait()
        gathers[nxt] = start_gather(nxt)
    for k in sorted(stores):
      stores.pop(k).wait()

  return emb_kernel


def kernel(idx, targets, wte, wpe):
  idx2 = idx.astype(jnp.int32).reshape(BT // CHUNK, CHUNK)
  x = _make_kernel()(idx2, wte, wpe)
  return x.reshape(B, T, D), targets


# speedup vs baseline: 1.1395x; 1.0694x over previous
import functools

import jax
import jax.numpy as jnp
from jax import lax
from jax.experimental import pallas as pl
from jax.experimental.pallas import tpu as pltpu
from jax.experimental.pallas import tpu_sc as plsc

VOCAB = 50304
D = 768
B, T = 4, 2048
BT = B * T

NC, NS, L = 2, 16, 16
NW = NC * NS
T_PER_W = T // NW
CHUNK = 32
SUB = T_PER_W // CHUNK
NCHUNK = B * SUB
NBUF = 3


@functools.cache
def _make_kernel():
  mesh = plsc.VectorSubcoreMesh(core_axis_name="c", subcore_axis_name="s")

  @functools.partial(
      pl.kernel,
      mesh=mesh,
      out_type=jax.ShapeDtypeStruct((BT, D), jnp.float32),
      scratch_types=[
          pltpu.VMEM((NCHUNK, CHUNK), jnp.int32),
          pltpu.VMEM((T_PER_W, D), jnp.float32),
      ] + [pltpu.VMEM((CHUNK, D), jnp.float32) for _ in range(NBUF)]
        + [pltpu.SemaphoreType.DMA for _ in range(2 * NBUF)],
  )
  def emb_kernel(idx_hbm, wte_hbm, wpe_hbm, out_hbm, idx_v, wpe_v,
                 r0, r1, r2, g0, g1, g2, s0, s1, s2):
    rows = (r0, r1, r2)
    gsem = (g0, g1, g2)
    ssem = (s0, s1, s2)
    wid = lax.axis_index("s") * NC + lax.axis_index("c")
    t0 = wid * T_PER_W

    for b in range(B):
      pltpu.sync_copy(idx_hbm.at[pl.ds(b * (T // CHUNK) + wid * SUB, SUB)],
                      idx_v.at[pl.ds(b * SUB, SUB)])
    pltpu.sync_copy(wpe_hbm.at[pl.ds(t0, T_PER_W)], wpe_v)

    def start_gather(k):
      p = k % NBUF
      return pltpu.async_copy(wte_hbm.at[idx_v.at[k]], rows[p], gsem[p])

    gathers = {k: start_gather(k) for k in range(min(NBUF - 1, NCHUNK))}
    stores = {}
    for k in range(NCHUNK):
      p = k % NBUF
      b, s = divmod(k, SUB)
      gathers.pop(k).wait()

      def add_row(i, _, s=s, p=p):
        for j in range(D // L):
          sl = pl.ds(j * L, L)
          plsc.addupdate(rows[p].at[i, sl], wpe_v[s * CHUNK + i, sl])
        return 0

      lax.fori_loop(0, CHUNK, add_row, 0)
      row0 = b * T + t0 + s * CHUNK
      stores[k] = pltpu.async_copy(rows[p], out_hbm.at[pl.ds(row0, CHUNK)],
                                   ssem[p])
      nxt = k + NBUF - 1
      if nxt < NCHUNK:
        prev = nxt - NBUF
        if prev >= 0:
          stores.pop(prev).wait()
        gathers[nxt] = start_gather(nxt)
    for k in sorted(stores):
      stores.pop(k).wait()

  return emb_kernel


def kernel(idx, targets, wte, wpe):
  idx2 = idx.astype(jnp.int32).reshape(BT // CHUNK, CHUNK)
  x = _make_kernel()(idx2, wte, wpe)
  return x.reshape(B, T, D), targets
